# baseline (speedup 1.0000x reference)
import functools
import jax
import jax.numpy as jnp
from jax import lax
from jax.experimental import pallas as pl
from jax.experimental.pallas import tpu as pltpu
from jax.experimental.pallas import tpu_sc as plsc

B, C, H, W = 16, 256, 32, 32
HW = H * W
K = 1024
HB = B // 2
NTOT = HB * HW
NC, NS = 2, 16
NW = NC * NS
ROWS_PER_W = NTOT // NW
CH = 128
NCHUNK = ROWS_PER_W // CH


def _tc_body(xb_ref, bank_ref, idx_ref, pn_ref):
    @pl.when(pl.program_id(0) == 0)
    def _():
        bank = bank_ref[...]
        pnorm = jnp.sqrt(jnp.sum(bank * bank, axis=1, keepdims=True))
        pn_ref[...] = bank / jnp.maximum(pnorm, 1e-12)

    xb = xb_ref[0]
    xnorm = jnp.sqrt(jnp.sum(xb * xb, axis=0, keepdims=True))
    xn = xb / jnp.maximum(xnorm, 1e-12)
    sims = jax.lax.dot_general(
        pn_ref[...], xn, (((1,), (0,)), ((), ())),
        preferred_element_type=jnp.float32)
    idx_ref[0] = jnp.argmax(sims, axis=0)[None, :].astype(jnp.int32)


def _sc_gather_body(table_hbm, idx_hbm, out_hbm, idx_v, rows_v, sem):
    wid = lax.axis_index("s") * NC + lax.axis_index("c")
    base = wid * ROWS_PER_W
    for j in range(NCHUNK):
        off = base + j * CH
        pltpu.sync_copy(idx_hbm.at[pl.ds(off, CH)], idx_v)
        pltpu.async_copy(table_hbm.at[idx_v], rows_v, sem).wait()
        pltpu.sync_copy(rows_v, out_hbm.at[pl.ds(off, CH)])


def _tc_indices(xb, prototype_bank, half):
    return pl.pallas_call(
        _tc_body,
        grid=(HB,),
        in_specs=[
            pl.BlockSpec((1, C, HW), lambda b: (b + half * HB, 0, 0)),
            pl.BlockSpec((K, C), lambda b: (0, 0)),
        ],
        out_specs=pl.BlockSpec((1, 1, HW), lambda b: (b, 0, 0)),
        out_shape=jax.ShapeDtypeStruct((HB, 1, HW), jnp.int32),
        scratch_shapes=[pltpu.VMEM((K, C), jnp.float32)],
    )(xb, prototype_bank)


def kernel(x, prototype_bank):
    xb = x.reshape(B, C, HW)

    mesh = plsc.VectorSubcoreMesh(core_axis_name="c", subcore_axis_name="s")
    sc_gather = functools.partial(
        pl.kernel,
        mesh=mesh,
        out_type=jax.ShapeDtypeStruct((NTOT, C), jnp.float32),
        scratch_types=[
            pltpu.VMEM((CH,), jnp.int32),
            pltpu.VMEM((CH, C), jnp.float32),
            pltpu.SemaphoreType.DMA,
        ],
    )(_sc_gather_body)

    idx_a = _tc_indices(xb, prototype_bank, 0)
    rows_a = sc_gather(prototype_bank, idx_a.reshape(NTOT))
    idx_b = _tc_indices(xb, prototype_bank, 1)
    rows_b = sc_gather(prototype_bank, idx_b.reshape(NTOT))

    recon_a = rows_a.reshape(HB, HW, C).transpose(0, 2, 1)
    recon_b = rows_b.reshape(HB, HW, C).transpose(0, 2, 1)
    recon = jnp.concatenate([recon_a, recon_b], axis=0).reshape(B, C, H, W)
    idx = jnp.concatenate([idx_a, idx_b], axis=0).reshape(B, HW)
    return recon, idx


# speedup vs baseline: 1.1595x; 1.1595x over previous
import functools
import jax
import jax.numpy as jnp
from jax import lax
from jax.experimental import pallas as pl
from jax.experimental.pallas import tpu as pltpu
from jax.experimental.pallas import tpu_sc as plsc

B, C, H, W = 16, 256, 32, 32
HW = H * W
K = 1024
NTOT = B * HW
NC, NS = 2, 16
NW = NC * NS
ROWS_PER_W = NTOT // NW
CH = 128
NCHUNK = ROWS_PER_W // CH


def _tc_body(xb_ref, bank_ref, idx_ref, pn_ref):
    @pl.when(pl.program_id(0) == 0)
    def _():
        bank = bank_ref[...]
        pnorm = jnp.sqrt(jnp.sum(bank * bank, axis=1, keepdims=True))
        pn_ref[...] = bank / jnp.maximum(pnorm, 1e-12)

    xb = xb_ref[0]
    xnorm = jnp.sqrt(jnp.sum(xb * xb, axis=0, keepdims=True))
    xn = xb / jnp.maximum(xnorm, 1e-12)
    sims = jax.lax.dot_general(
        pn_ref[...], xn, (((1,), (0,)), ((), ())),
        preferred_element_type=jnp.float32)
    idx_ref[0] = jnp.argmax(sims, axis=0)[None, :].astype(jnp.int32)


def _sc_gather_body(table_hbm, idx_hbm, out_hbm,
                    idx0, idx1, rows0, rows1, g0, g1, w0, w1):
    wid = lax.axis_index("s") * NC + lax.axis_index("c")
    base = wid * ROWS_PER_W
    idx_bufs = (idx0, idx1)
    row_bufs = (rows0, rows1)
    gsems = (g0, g1)
    wsems = (w0, w1)
    gathers = [None] * NCHUNK
    writes = [None] * NCHUNK
    for j in range(NCHUNK):
        p = j % 2
        if j >= 2:
            writes[j - 2].wait()
        pltpu.sync_copy(idx_hbm.at[pl.ds(base + j * CH, CH)], idx_bufs[p])
        gathers[j] = pltpu.async_copy(
            table_hbm.at[idx_bufs[p]], row_bufs[p], gsems[p])
        if j >= 1:
            q = (j - 1) % 2
            gathers[j - 1].wait()
            writes[j - 1] = pltpu.async_copy(
                row_bufs[q], out_hbm.at[pl.ds(base + (j - 1) * CH, CH)],
                wsems[q])
    gathers[NCHUNK - 1].wait()
    writes[NCHUNK - 1] = pltpu.async_copy(
        row_bufs[(NCHUNK - 1) % 2],
        out_hbm.at[pl.ds(base + (NCHUNK - 1) * CH, CH)],
        wsems[(NCHUNK - 1) % 2])
    if NCHUNK >= 2:
        writes[NCHUNK - 2].wait()
    writes[NCHUNK - 1].wait()


def kernel(x, prototype_bank):
    xb = x.reshape(B, C, HW)
    idx = pl.pallas_call(
        _tc_body,
        grid=(B,),
        in_specs=[
            pl.BlockSpec((1, C, HW), lambda b: (b, 0, 0)),
            pl.BlockSpec((K, C), lambda b: (0, 0)),
        ],
        out_specs=pl.BlockSpec((1, 1, HW), lambda b: (b, 0, 0)),
        out_shape=jax.ShapeDtypeStruct((B, 1, HW), jnp.int32),
        scratch_shapes=[pltpu.VMEM((K, C), jnp.float32)],
    )(xb, prototype_bank)

    mesh = plsc.VectorSubcoreMesh(core_axis_name="c", subcore_axis_name="s")
    sc_gather = functools.partial(
        pl.kernel,
        mesh=mesh,
        out_type=jax.ShapeDtypeStruct((NTOT, C), jnp.float32),
        scratch_types=[
            pltpu.VMEM((CH,), jnp.int32),
            pltpu.VMEM((CH,), jnp.int32),
            pltpu.VMEM((CH, C), jnp.float32),
            pltpu.VMEM((CH, C), jnp.float32),
            pltpu.SemaphoreType.DMA,
            pltpu.SemaphoreType.DMA,
            pltpu.SemaphoreType.DMA,
            pltpu.SemaphoreType.DMA,
        ],
    )(_sc_gather_body)
    rows = sc_gather(prototype_bank, idx.reshape(NTOT))

    recon = rows.reshape(B, HW, C).transpose(0, 2, 1).reshape(B, C, H, W)
    return recon, idx.reshape(B, HW)


# speedup vs baseline: 1.1598x; 1.0002x over previous
import functools
import jax
import jax.numpy as jnp
from jax import lax
from jax.experimental import pallas as pl
from jax.experimental.pallas import tpu as pltpu
from jax.experimental.pallas import tpu_sc as plsc

B, C, H, W = 16, 256, 32, 32
HW = H * W
K = 1024
NTOT = B * HW
NC, NS = 2, 16
NW = NC * NS
ROWS_PER_W = NTOT // NW
CH = 128
NCHUNK = ROWS_PER_W // CH


def _tc_body(xb_ref, bank_ref, idx_ref, pn_ref):
    @pl.when(pl.program_id(0) == 0)
    def _():
        bank = bank_ref[...]
        pnorm = jnp.sqrt(jnp.sum(bank * bank, axis=1, keepdims=True))
        pn_ref[...] = bank / jnp.maximum(pnorm, 1e-12)

    xb = xb_ref[0]
    xnorm = jnp.sqrt(jnp.sum(xb * xb, axis=0, keepdims=True))
    xn = xb / jnp.maximum(xnorm, 1e-12)
    sims = jax.lax.dot_general(
        pn_ref[...], xn, (((1,), (0,)), ((), ())),
        preferred_element_type=jnp.float32)
    idx_ref[0] = jnp.argmax(sims, axis=0)[None, :].astype(jnp.int32)


def _sc_gather_body(table_hbm, idx_hbm, out_hbm,
                    idx_all, rows0, rows1, g0, g1, w0, w1):
    wid = lax.axis_index("s") * NC + lax.axis_index("c")
    base = wid * ROWS_PER_W
    pltpu.sync_copy(idx_hbm.at[pl.ds(base, ROWS_PER_W)], idx_all)
    row_bufs = (rows0, rows1)
    gsems = (g0, g1)
    wsems = (w0, w1)
    gathers = [None] * NCHUNK
    writes = [None] * NCHUNK
    for j in range(NCHUNK):
        p = j % 2
        if j >= 2:
            writes[j - 2].wait()
        gathers[j] = pltpu.async_copy(
            table_hbm.at[idx_all.at[pl.ds(j * CH, CH)]], row_bufs[p],
            gsems[p])
        if j >= 1:
            q = (j - 1) % 2
            gathers[j - 1].wait()
            writes[j - 1] = pltpu.async_copy(
                row_bufs[q], out_hbm.at[pl.ds(base + (j - 1) * CH, CH)],
                wsems[q])
    gathers[NCHUNK - 1].wait()
    writes[NCHUNK - 1] = pltpu.async_copy(
        row_bufs[(NCHUNK - 1) % 2],
        out_hbm.at[pl.ds(base + (NCHUNK - 1) * CH, CH)],
        wsems[(NCHUNK - 1) % 2])
    if NCHUNK >= 2:
        writes[NCHUNK - 2].wait()
    writes[NCHUNK - 1].wait()


def kernel(x, prototype_bank):
    xb = x.reshape(B, C, HW)
    idx = pl.pallas_call(
        _tc_body,
        grid=(B,),
        in_specs=[
            pl.BlockSpec((1, C, HW), lambda b: (b, 0, 0)),
            pl.BlockSpec((K, C), lambda b: (0, 0)),
        ],
        out_specs=pl.BlockSpec((1, 1, HW), lambda b: (b, 0, 0)),
        out_shape=jax.ShapeDtypeStruct((B, 1, HW), jnp.int32),
        scratch_shapes=[pltpu.VMEM((K, C), jnp.float32)],
    )(xb, prototype_bank)

    mesh = plsc.VectorSubcoreMesh(core_axis_name="c", subcore_axis_name="s")
    sc_gather = functools.partial(
        pl.kernel,
        mesh=mesh,
        out_type=jax.ShapeDtypeStruct((NTOT, C), jnp.float32),
        scratch_types=[
            pltpu.VMEM((ROWS_PER_W,), jnp.int32),
            pltpu.VMEM((CH, C), jnp.float32),
            pltpu.VMEM((CH, C), jnp.float32),
            pltpu.SemaphoreType.DMA,
            pltpu.SemaphoreType.DMA,
            pltpu.SemaphoreType.DMA,
            pltpu.SemaphoreType.DMA,
        ],
    )(_sc_gather_body)
    rows = sc_gather(prototype_bank, idx.reshape(NTOT))

    recon = rows.reshape(B, HW, C).transpose(0, 2, 1).reshape(B, C, H, W)
    return recon, idx.reshape(B, HW)
